# baseline (speedup 1.0000x reference)
import functools

import jax
import jax.numpy as jnp
from jax import lax
from jax.experimental import pallas as pl
from jax.experimental.pallas import tpu as pltpu
from jax.experimental.pallas import tpu_sc as plsc

_N_EDGES = 160000
_D = 256
_NC = 2
_NS = 16
_NW = _NC * _NS
_TOTAL = 2 * _N_EDGES
_PER_W = _TOTAL // _NW
_C = 80
_N_CHUNKS = _PER_W // _C
_G = _C // 16


def _make_kernel():
    mesh = plsc.VectorSubcoreMesh(core_axis_name="c", subcore_axis_name="s")

    @functools.partial(
        pl.kernel,
        mesh=mesh,
        out_type=jax.ShapeDtypeStruct((_TOTAL,), jnp.float32),
        compiler_params=pltpu.CompilerParams(
            use_tc_tiling_on_sc=False, needs_layout_passes=False),
        scratch_types=[
            pltpu.VMEM((_C,), jnp.int32),
            pltpu.VMEM((_C,), jnp.int32),
            pltpu.VMEM((_C, _D), jnp.float32),
            pltpu.VMEM((_C, _D), jnp.float32),
            pltpu.VMEM((_PER_W,), jnp.float32),
            pltpu.SemaphoreType.DMA,
            pltpu.SemaphoreType.DMA,
        ],
    )
    def decode(z_hbm, src_hbm, dst_hbm, out_hbm,
               sidx, didx, srows, drows, outv, sem_s, sem_d):
        wid = lax.axis_index("s") * _NC + lax.axis_index("c")
        base = wid * _PER_W
        lanes = lax.iota(jnp.int32, 16)

        def chunk_body(i, carry):
            off = base + i * _C
            pltpu.sync_copy(src_hbm.at[pl.ds(off, _C)], sidx)
            pltpu.sync_copy(dst_hbm.at[pl.ds(off, _C)], didx)
            cp_s = pltpu.async_copy(z_hbm.at[sidx], srows, sem_s)
            cp_d = pltpu.async_copy(z_hbm.at[didx], drows, sem_d)
            cp_s.wait()
            cp_d.wait()
            for g in range(_G):
                rows = lanes + (g * 16)

                def feat_body(j, acc):
                    col = jnp.full((16,), 0, jnp.int32) + j
                    a = plsc.load_gather(srows, [rows, col])
                    b = plsc.load_gather(drows, [rows, col])
                    return acc + a * b

                acc = lax.fori_loop(0, _D, feat_body,
                                    jnp.zeros((16,), jnp.float32), unroll=8)
                outv[pl.ds(i * _C + g * 16, 16)] = acc
            return carry

        lax.fori_loop(0, _N_CHUNKS, chunk_body, 0)
        pltpu.sync_copy(outv, out_hbm.at[pl.ds(base, _PER_W)])

    return decode


_decode = _make_kernel()


def kernel(z, edge_index_pos, edge_index_neg):
    src = jnp.concatenate(
        [edge_index_pos[0], edge_index_neg[0]]).astype(jnp.int32)
    dst = jnp.concatenate(
        [edge_index_pos[1], edge_index_neg[1]]).astype(jnp.int32)
    scores = _decode(z, src, dst)
    return scores[:_N_EDGES], scores[_N_EDGES:]


# speedup vs baseline: 1.1442x; 1.1442x over previous
import functools

import jax
import jax.numpy as jnp
from jax import lax
from jax.experimental import pallas as pl
from jax.experimental.pallas import tpu as pltpu
from jax.experimental.pallas import tpu_sc as plsc

_N_EDGES = 160000
_D = 256
_NC = 2
_NS = 16
_NW = _NC * _NS
_TOTAL = 2 * _N_EDGES
_PER_W = _TOTAL // _NW
_C = 80
_IDXW = 2 * _C
_N_CHUNKS = _PER_W // _C
_G = _C // 16


def _make_kernel():
    mesh = plsc.VectorSubcoreMesh(core_axis_name="c", subcore_axis_name="s")

    @functools.partial(
        pl.kernel,
        mesh=mesh,
        out_type=jax.ShapeDtypeStruct((_TOTAL,), jnp.float32),
        compiler_params=pltpu.CompilerParams(
            use_tc_tiling_on_sc=False, needs_layout_passes=False),
        scratch_types=[
            pltpu.VMEM((_PER_W * 2,), jnp.int32),
            pltpu.VMEM((_C, _D), jnp.float32),
            pltpu.VMEM((_C, _D), jnp.float32),
            pltpu.VMEM((_C, _D), jnp.float32),
            pltpu.VMEM((_C, _D), jnp.float32),
            pltpu.VMEM((_PER_W,), jnp.float32),
            pltpu.SemaphoreType.DMA,
            pltpu.SemaphoreType.DMA,
            pltpu.SemaphoreType.DMA,
            pltpu.SemaphoreType.DMA,
        ],
    )
    def decode(z_hbm, idx_hbm, out_hbm,
               idxv, sbuf0, dbuf0, sbuf1, dbuf1, outv,
               ss0, sd0, ss1, sd1):
        wid = lax.axis_index("s") * _NC + lax.axis_index("c")
        base = wid * _PER_W
        lanes = lax.iota(jnp.int32, 16)

        pltpu.sync_copy(idx_hbm.at[pl.ds(base * 2, _PER_W * 2)], idxv)

        def gather_pair(i, sb, db, ss, sd):
            o = i * _IDXW
            pltpu.async_copy(z_hbm.at[idxv.at[pl.ds(o, _C)]], sb, ss)
            pltpu.async_copy(z_hbm.at[idxv.at[pl.ds(o + _C, _C)]], db, sd)

        def wait_pair(i, sb, db, ss, sd):
            o = i * _IDXW
            pltpu.make_async_copy(
                z_hbm.at[idxv.at[pl.ds(o, _C)]], sb, ss).wait()
            pltpu.make_async_copy(
                z_hbm.at[idxv.at[pl.ds(o + _C, _C)]], db, sd).wait()

        def compute(i, sb, db):
            for g in range(_G):
                rows = lanes + (g * 16)

                def feat_body(j, acc):
                    col = jnp.zeros((16,), jnp.int32) + j
                    a = plsc.load_gather(sb, [rows, col])
                    b = plsc.load_gather(db, [rows, col])
                    return acc + a * b

                acc = lax.fori_loop(0, _D, feat_body,
                                    jnp.zeros((16,), jnp.float32), unroll=8)
                outv[pl.ds(i * _C + g * 16, 16)] = acc

        gather_pair(0, sbuf0, dbuf0, ss0, sd0)
        gather_pair(1, sbuf1, dbuf1, ss1, sd1)

        def pair_body(h, carry):
            i0 = 2 * h
            wait_pair(i0, sbuf0, dbuf0, ss0, sd0)
            compute(i0, sbuf0, dbuf0)
            gather_pair(i0 + 2, sbuf0, dbuf0, ss0, sd0)
            wait_pair(i0 + 1, sbuf1, dbuf1, ss1, sd1)
            compute(i0 + 1, sbuf1, dbuf1)

            @pl.when(i0 + 3 < _N_CHUNKS)
            def _():
                gather_pair(i0 + 3, sbuf1, dbuf1, ss1, sd1)

            return carry

        lax.fori_loop(0, (_N_CHUNKS - 1) // 2, pair_body, 0)
        last = _N_CHUNKS - 1
        wait_pair(last, sbuf0, dbuf0, ss0, sd0)
        compute(last, sbuf0, dbuf0)
        pltpu.sync_copy(outv, out_hbm.at[pl.ds(base, _PER_W)])

    return decode


_decode = _make_kernel()


def kernel(z, edge_index_pos, edge_index_neg):
    src = jnp.concatenate(
        [edge_index_pos[0], edge_index_neg[0]]).astype(jnp.int32)
    dst = jnp.concatenate(
        [edge_index_pos[1], edge_index_neg[1]]).astype(jnp.int32)
    both = jnp.stack([src, dst]).reshape(2, _NW, _N_CHUNKS, _C)
    both = both.transpose(1, 2, 0, 3).reshape(-1)
    scores = _decode(z, both)
    return scores[:_N_EDGES], scores[_N_EDGES:]
